# baseline (speedup 1.0000x reference)
import functools

import jax
import jax.numpy as jnp
import numpy as np
from jax import lax
from jax.experimental import pallas as pl
from jax.experimental.pallas import tpu as pltpu
from jax.experimental.pallas import tpu_sc as plsc

_SHAPE = (128, 32768)
_R, _C = _SHAPE
_TOTAL = _R * _C

_MASK_KEY = jax.random.key(42)
_MASK = np.asarray(
    jax.random.uniform(_MASK_KEY, _SHAPE, dtype=jnp.float32) > 0.5
)
_IDX_NP = np.flatnonzero(_MASK.ravel()).astype(np.int64)
_N = int(_IDX_NP.shape[0])

_NC, _NS = 2, 16
_NW = _NC * _NS
_ROWS_PER_W = _R // _NW

_BUF_ROWS = 272
_NXT_BASE = 256 * 128
_PRV_BASE = 264 * 128
_EDGE = 1024


_WBLK = 9


def _build_static():
    cnt = _MASK.sum(axis=1).astype(np.int64)
    off = np.concatenate([[0], np.cumsum(cnt)])[:-1]
    base = int(cnt.max()) + 7
    s = base + int((_N - base) % 8)
    s_idx = ((s + 31) // 32) * 32
    assert s_idx // 2 <= _WBLK * 1024
    words = np.zeros((_R, _WBLK * 1024), dtype=np.uint32)
    woffs = np.zeros((1024,), dtype=np.int32)
    for r in range(_R):
        if r < _R - 1:
            woff = int(off[r]) // 8 * 8
        else:
            woff = _N - s
        assert woff % 8 == 0 and woff + s <= _N
        assert woff <= off[r] and off[r] + cnt[r] <= woff + s
        src = _IDX_NP[woff:woff + s]
        srow = src // _C
        scol = src % _C
        local = np.zeros((s,), dtype=np.int64)
        cur = srow == r
        local[cur] = scol[cur]
        nxt = srow == r + 1
        assert np.all(scol[nxt] < _EDGE), r
        local[nxt] = _NXT_BASE + scol[nxt]
        prv = srow == r - 1
        assert np.all(scol[prv] >= _C - _EDGE), r
        local[prv] = _PRV_BASE + (scol[prv] - (_C - _EDGE))
        assert np.all(cur | nxt | prv), r
        assert local.max() < _BUF_ROWS * 128 <= 65536
        local = np.concatenate(
            [local, np.zeros((s_idx - s,), np.int64)]).astype(np.uint32)
        blocks = local.reshape(s_idx // 32, 32)
        words[r, :s_idx // 2] = (
            blocks[:, :16] | (blocks[:, 16:] << np.uint32(16))).reshape(-1)
        w = r // _ROWS_PER_W
        woffs[w * 16 + (r % _ROWS_PER_W)] = woff
    tab = np.concatenate([words.reshape(-1), woffs.view(np.uint32)])
    tab = tab.view(np.int32).reshape(_R * _WBLK + 1, 8, 128)
    return tab, s, s_idx


_TAB, _S, _S_IDX = _build_static()


def _sc_body(x4, tab_hbm, out_hbm, buf, w_buf, o_buf, s_buf, semi, semo):
    wid = lax.axis_index("s") * _NC + lax.axis_index("c")
    pltpu.sync_copy(tab_hbm.at[_R * _WBLK], s_buf)
    sv = s_buf[wid >> 3, pl.ds((wid & 7) * 16, 16)]

    def in_copies(j):
        b = j & 1
        r = wid * _ROWS_PER_W + j
        rp = jnp.maximum(r - 1, 0)
        rn = jnp.minimum(r + 1, _R - 1)
        hs = (
            pltpu.make_async_copy(
                x4.at[r >> 3, :, r & 7, :],
                buf.at[b, pl.ds(0, 256), :], semi.at[b]),
            pltpu.make_async_copy(
                x4.at[rn >> 3, pl.ds(0, 8), rn & 7, :],
                buf.at[b, pl.ds(256, 8), :], semi.at[b]),
            pltpu.make_async_copy(
                x4.at[rp >> 3, pl.ds(248, 8), rp & 7, :],
                buf.at[b, pl.ds(264, 8), :], semi.at[b]),
            pltpu.make_async_copy(
                tab_hbm.at[pl.ds(r * _WBLK, _WBLK)],
                w_buf.at[b], semi.at[b]),
        )
        for h in hs:
            h.start()
        return hs

    in_h = [None, None]
    out_h = [None, None]
    in_h[0] = in_copies(0)
    for j in range(_ROWS_PER_W):
        b = j & 1
        if j + 1 < _ROWS_PER_W:
            in_h[(j + 1) & 1] = in_copies(j + 1)
        for h in in_h[b]:
            h.wait()
        if out_h[b] is not None:
            out_h[b].wait()
            out_h[b] = None
        bf, ob, wb = buf.at[b], o_buf.at[b], w_buf.at[b]

        @plsc.parallel_loop(0, _S_IDX // 32, unroll=8)
        def _(i):
            o = i * 16
            v = wb[o >> 10, (o >> 7) & 7, pl.ds(o & 127, 16)]
            lo = jnp.bitwise_and(v, jnp.int32(0xFFFF))
            hi = lax.shift_right_logical(v, 16)
            ob[pl.ds(i * 32, 16)] = plsc.load_gather(
                bf, [lax.shift_right_logical(lo, 7),
                     jnp.bitwise_and(lo, jnp.int32(127))])
            ob[pl.ds(i * 32 + 16, 16)] = plsc.load_gather(
                bf, [lax.shift_right_logical(hi, 7),
                     jnp.bitwise_and(hi, jnp.int32(127))])

        woff = pl.multiple_of(sv[j], 8)
        ho = pltpu.make_async_copy(
            ob.at[pl.ds(0, _S)], out_hbm.at[pl.ds(woff, _S)], semo.at[b])
        ho.start()
        out_h[b] = ho

    for h in out_h:
        if h is not None:
            h.wait()


@functools.cache
def _sc_gather():
    return pl.kernel(
        _sc_body,
        out_type=jax.ShapeDtypeStruct((_N,), jnp.float32),
        mesh=plsc.VectorSubcoreMesh(
            core_axis_name="c", subcore_axis_name="s",
            num_cores=_NC, num_subcores=_NS,
        ),
        scratch_types=[
            pltpu.VMEM((2, _BUF_ROWS, 128), jnp.float32),
            pltpu.VMEM((2, _WBLK, 8, 128), jnp.int32),
            pltpu.VMEM((2, _S_IDX), jnp.float32),
            pltpu.VMEM((8, 128), jnp.int32),
            pltpu.SemaphoreType.DMA((2,)),
            pltpu.SemaphoreType.DMA((2,)),
        ],
        compiler_params=pltpu.CompilerParams(
            use_tc_tiling_on_sc=False,
            needs_layout_passes=False,
        ),
    )


@jax.jit
def kernel(x):
    x4 = jnp.transpose(x.reshape(16, 8, 256, 128), (0, 2, 1, 3))
    return _sc_gather()(x4, jnp.asarray(_TAB))


# speedup vs baseline: 1.3550x; 1.3550x over previous
import functools

import jax
import jax.numpy as jnp
import numpy as np
from jax import lax
from jax.experimental import pallas as pl
from jax.experimental.pallas import tpu as pltpu
from jax.experimental.pallas import tpu_sc as plsc

_SHAPE = (128, 32768)
_TOTAL = _SHAPE[0] * _SHAPE[1]

_MASK_KEY = jax.random.key(42)
_MASK = np.asarray(
    jax.random.uniform(_MASK_KEY, _SHAPE, dtype=jnp.float32) > 0.5
).ravel()
_IDX_NP = np.flatnonzero(_MASK).astype(np.int64)
_N = int(_IDX_NP.shape[0])

_NC, _NS = 2, 16
_NW = _NC * _NS
_N_CHUNKS = 256
_CHUNKS_PER_W = _N_CHUNKS // _NW
_COUT = 8192
assert _N_CHUNKS * _COUT >= _N


def _build_static():
    starts = np.zeros((_N_CHUNKS,), dtype=np.int32)
    locals_ = np.zeros((_N_CHUNKS, _COUT), dtype=np.int64)
    spans = np.zeros((_N_CHUNKS,), dtype=np.int64)
    for c in range(_N_CHUNKS):
        o0 = c * _COUT
        o1 = min(_N, o0 + _COUT)
        chunk = _IDX_NP[o0:o1]
        if chunk.size < _COUT:
            chunk = np.concatenate(
                [chunk, np.full((_COUT - chunk.size,), chunk[-1], np.int64)]
            )
        s = (int(chunk[0]) // 8) * 8
        starts[c] = s
        locals_[c] = chunk - s
        spans[c] = chunk[-1] - s + 1
    w = int(spans.max())
    w = ((w + 15) // 16) * 16
    over = starts > _TOTAL - w
    locals_[over] += (starts[over] - (_TOTAL - w))[:, None]
    starts[over] = _TOTAL - w
    assert locals_.min() >= 0 and locals_.max() < w <= 65536
    blocks = locals_.reshape(_N_CHUNKS, _COUT // 32, 32).astype(np.uint32)
    words = (blocks[:, :, :16] | (blocks[:, :, 16:] << np.uint32(16)))
    words = words.reshape(_N_CHUNKS, _COUT // 2).view(np.int32)
    starts = np.concatenate([starts, np.zeros((16,), np.int32)])
    return words, starts, w


_WORDS, _STARTS, _W = _build_static()


_TAIL = _N - (_N_CHUNKS - 1) * _COUT


def _sc_body(x_hbm, w_hbm, s_hbm, out_hbm, x_buf, w_buf, o_buf, s_buf,
             semx, semw, semo):
    wid = lax.axis_index("s") * _NC + lax.axis_index("c")
    pltpu.sync_copy(s_hbm, s_buf)
    sv = s_buf[pl.ds(wid * _CHUNKS_PER_W, 16)]

    def in_copies(j):
        b = j & 1
        c = wid * _CHUNKS_PER_W + j
        st = pl.multiple_of(sv[j], 8)
        hx = pltpu.make_async_copy(
            x_hbm.at[pl.ds(st, _W)], x_buf.at[b], semx.at[b])
        hw = pltpu.make_async_copy(w_hbm.at[c], w_buf.at[b], semw.at[b])
        hx.start()
        hw.start()
        return hx, hw

    in_h = [None, None]
    out_h = [None, None]
    in_h[0] = in_copies(0)
    for j in range(_CHUNKS_PER_W):
        b = j & 1
        c = wid * _CHUNKS_PER_W + j
        if j + 1 < _CHUNKS_PER_W:
            in_h[(j + 1) & 1] = in_copies(j + 1)
        hx, hw = in_h[b]
        hx.wait()
        hw.wait()
        if out_h[b] is not None:
            out_h[b].wait()
            out_h[b] = None
        xb, wb, ob = x_buf.at[b], w_buf.at[b], o_buf.at[b]

        @plsc.parallel_loop(0, _COUT // 32, unroll=8)
        def _(i):
            v = wb[pl.ds(i * 16, 16)]
            lo = jnp.bitwise_and(v, jnp.int32(0xFFFF))
            hi = lax.shift_right_logical(v, 16)
            ob[pl.ds(i * 32, 16)] = plsc.load_gather(xb, [lo])
            ob[pl.ds(i * 32 + 16, 16)] = plsc.load_gather(xb, [hi])
        if j + 1 < _CHUNKS_PER_W:
            ho = pltpu.make_async_copy(
                ob, out_hbm.at[pl.ds(c * _COUT, _COUT)], semo.at[b])
            ho.start()
            out_h[b] = ho
        else:
            @pl.when(wid != _NW - 1)
            def _():
                pltpu.sync_copy(ob, out_hbm.at[pl.ds(c * _COUT, _COUT)])

            @pl.when(wid == _NW - 1)
            def _():
                pltpu.sync_copy(
                    ob.at[pl.ds(0, _TAIL)],
                    out_hbm.at[pl.ds(c * _COUT, _TAIL)])

    for h in out_h:
        if h is not None:
            h.wait()


@functools.cache
def _sc_gather():
    return pl.kernel(
        _sc_body,
        out_type=jax.ShapeDtypeStruct((_N,), jnp.float32),
        mesh=plsc.VectorSubcoreMesh(
            core_axis_name="c", subcore_axis_name="s",
            num_cores=_NC, num_subcores=_NS,
        ),
        scratch_types=[
            pltpu.VMEM((2, _W), jnp.float32),
            pltpu.VMEM((2, _COUT // 2), jnp.int32),
            pltpu.VMEM((2, _COUT), jnp.float32),
            pltpu.VMEM((_N_CHUNKS + 16,), jnp.int32),
            pltpu.SemaphoreType.DMA((2,)),
            pltpu.SemaphoreType.DMA((2,)),
            pltpu.SemaphoreType.DMA((2,)),
        ],
        compiler_params=pltpu.CompilerParams(
            use_tc_tiling_on_sc=False,
            needs_layout_passes=False,
        ),
    )


@jax.jit
def kernel(x):
    return _sc_gather()(x.reshape(-1), jnp.asarray(_WORDS), jnp.asarray(_STARTS))
